# Initial kernel scaffold; baseline (speedup 1.0000x reference)
import jax
import jax.numpy as jnp
from jax.experimental import pallas as pl


def kernel(x, offset, edge, W1, b1, W2, b2, W_ih, W_hh, b_ih, b_hh):
    raise NotImplementedError("write your pallas kernel here")


# speedup vs baseline: 6.3417x; 6.3417x over previous
import functools

import jax
import jax.numpy as jnp
from jax import lax
from jax.experimental import pallas as pl
from jax.experimental.pallas import tpu as pltpu
from jax.experimental.pallas import tpu_sc as plsc

N = 50000
E = 800000
IN = 128
H = 64
FQ = 16
NQ = 4
BATCH = 64

NT = 16
BLK = 128
NB = 392
EPAD = NT * NB * BLK
MAC = 56
NMAC = NB // MAC
RING = 4
ROWS_T = N // NT
DR_A = 3128
DR_B = N - (NT - 1) * DR_A
ZB = 625
NPAD_C = 51200
CT = NPAD_C // NT
ZBC = 640

_F32 = jnp.float32
_HI = lax.Precision.HIGHEST


def _mesh():
    return plsc.VectorSubcoreMesh(core_axis_name="c", subcore_axis_name="s")


def _sc_msum(x2q, src2, dst2):
    @functools.partial(
        pl.kernel,
        out_type=jax.ShapeDtypeStruct((NQ, N, FQ), _F32),
        mesh=_mesh(),
        compiler_params=pltpu.CompilerParams(use_tc_tiling_on_sc=False),
        scratch_types=[
            pltpu.VMEM((MAC, BLK), jnp.int32),
            pltpu.VMEM((MAC, BLK), jnp.int32),
            pltpu.VMEM((RING, BLK, FQ), _F32),
            pltpu.VMEM((ZB, FQ), _F32),
            pltpu.VMEM_SHARED((N + 16, FQ), _F32),
            pltpu.SemaphoreType.DMA,
            pltpu.SemaphoreType.DMA,
            pltpu.SemaphoreType.DMA,
            pltpu.SemaphoreType.DMA,
        ],
    )
    def k(tq_hbm, src_hbm, dst_hbm, out_hbm, sidx, didx, rows, zbuf,
          acc, sm0, sm1, sm2, sm3):
        c = lax.axis_index("c")
        s = lax.axis_index("s")
        sems = (sm0, sm1, sm2, sm3)

        z16 = jnp.zeros((16,), _F32)

        def zfill(i, carry):
            zbuf[i, :] = z16
            return carry

        lax.fori_loop(0, ZB, zfill, 0)

        def one_pass(q):
            base = s * ROWS_T
            for kk in range(ROWS_T // ZB):
                pltpu.sync_copy(zbuf, acc.at[pl.ds(base + kk * ZB, ZB)])

            @pl.when(s == 0)
            def _():
                pltpu.sync_copy(zbuf.at[pl.ds(0, 16)], acc.at[pl.ds(N, 16)])

            plsc.subcore_barrier()

            table = tq_hbm.at[q]

            def mac_step(m, carry):
                row0 = s * NB + m * MAC
                pltpu.sync_copy(src_hbm.at[pl.ds(row0, MAC)], sidx)
                pltpu.sync_copy(dst_hbm.at[pl.ds(row0, MAC)], didx)
                for b in range(RING):
                    pltpu.async_copy(table.at[sidx.at[b]], rows.at[b], sems[b])

                def step(g, carry2):
                    for b in range(RING):
                        j = g * RING + b
                        pltpu.make_async_copy(
                            table.at[sidx.at[j]], rows.at[b], sems[b]).wait()
                        pltpu.sync_copy(rows.at[b], acc.at[didx.at[j]],
                                        add=True)

                        @pl.when(g < MAC // RING - 1)
                        def _():
                            pltpu.async_copy(table.at[sidx.at[j + RING]],
                                             rows.at[b], sems[b])
                    return carry2

                lax.fori_loop(0, MAC // RING, step, 0)
                return carry

            lax.fori_loop(0, NMAC, mac_step, 0)
            plsc.subcore_barrier()

            out_slice = out_hbm.at[q]

            @pl.when(s < NT - 1)
            def _():
                d = pl.ds(pl.multiple_of(s * DR_A, 8), DR_A)
                pltpu.sync_copy(acc.at[d], out_slice.at[d])

            @pl.when(s == NT - 1)
            def _():
                d = pl.ds((NT - 1) * DR_A, DR_B)
                pltpu.sync_copy(acc.at[d], out_slice.at[d])

            plsc.subcore_barrier()

        @pl.when(c == 0)
        def _():
            one_pass(0)
            one_pass(1)

        @pl.when(c == 1)
        def _():
            one_pass(2)
            one_pass(3)

    return k(x2q, src2, dst2)


def _sc_counts(dst2):
    @functools.partial(
        pl.kernel,
        out_type=jax.ShapeDtypeStruct((2, NPAD_C), _F32),
        mesh=_mesh(),
        compiler_params=pltpu.CompilerParams(use_tc_tiling_on_sc=False),
        scratch_types=[
            pltpu.VMEM((MAC, BLK), jnp.int32),
            pltpu.VMEM((BLK,), _F32),
            pltpu.VMEM((ZBC,), _F32),
            pltpu.VMEM_SHARED((NPAD_C,), _F32),
            pltpu.SemaphoreType.DMA,
            pltpu.SemaphoreType.DMA,
            pltpu.SemaphoreType.DMA,
            pltpu.SemaphoreType.DMA,
        ],
    )
    def k(dst_hbm, out_hbm, didx, ones_v, zbuf, cacc, sm0, sm1, sm2, sm3):
        c = lax.axis_index("c")
        s = lax.axis_index("s")
        sems = (sm0, sm1, sm2, sm3)

        one16 = jnp.ones((16,), _F32)
        z16 = jnp.zeros((16,), _F32)
        for i in range(BLK // 16):
            ones_v[pl.ds(16 * i, 16)] = one16

        def zf(i, carry):
            zbuf[pl.ds(i * 16, 16)] = z16
            return carry

        lax.fori_loop(0, ZBC // 16, zf, 0)
        base = s * CT
        for kk in range(CT // ZBC):
            pltpu.sync_copy(zbuf, cacc.at[pl.ds(base + kk * ZBC, ZBC)])
        plsc.subcore_barrier()

        def run(ci):
            def mac_step(m, carry):
                row0 = s * NB + m * MAC
                pltpu.sync_copy(dst_hbm.at[ci].at[pl.ds(row0, MAC)], didx)

                def step(g, carry2):
                    for b in range(RING):
                        j = g * RING + b

                        @pl.when(g > 0)
                        def _():
                            pltpu.make_async_copy(
                                ones_v, cacc.at[didx.at[j]], sems[b]).wait()

                        pltpu.async_copy(ones_v, cacc.at[didx.at[j]], sems[b],
                                         add=True)
                    return carry2

                lax.fori_loop(0, MAC // RING, step, 0)
                for b in range(RING):
                    pltpu.make_async_copy(
                        ones_v, cacc.at[didx.at[b]], sems[b]).wait()
                return carry

            lax.fori_loop(0, NMAC, mac_step, 0)

        @pl.when(c == 0)
        def _():
            run(0)

        @pl.when(c == 1)
        def _():
            run(1)

        plsc.subcore_barrier()
        dr = pl.ds(s * CT, CT)

        @pl.when(c == 0)
        def _():
            pltpu.sync_copy(cacc.at[dr], out_hbm.at[0].at[dr])

        @pl.when(c == 1)
        def _():
            pltpu.sync_copy(cacc.at[dr], out_hbm.at[1].at[dr])

    return k(dst2)


BN = 1000


def _dot(a, b):
    return jnp.dot(a, b, preferred_element_type=_F32, precision=_HI)


def _split_q(y, o_ref):
    for q in range(NQ):
        o_ref[q] = y[:, q * FQ:(q + 1) * FQ]


def _cat_q(ref):
    return jnp.concatenate([ref[q] for q in range(NQ)], axis=1)


def _lin1_body(x_ref, w_ref, b_ref, o_ref):
    y = _dot(x_ref[...], w_ref[...]) + b_ref[...]
    _split_q(y, o_ref)


def _lin1(x, w1t, b1r):
    return pl.pallas_call(
        _lin1_body,
        grid=(N // BN,),
        in_specs=[
            pl.BlockSpec((BN, IN), lambda i: (i, 0)),
            pl.BlockSpec((IN, H), lambda i: (0, 0)),
            pl.BlockSpec((1, H), lambda i: (0, 0)),
        ],
        out_specs=pl.BlockSpec((NQ, BN, FQ), lambda i: (0, i, 0)),
        out_shape=jax.ShapeDtypeStruct((NQ, N, FQ), _F32),
    )(x, w1t, b1r)


def _gru_core(x2_ref, ms_ref, cnt_ref, wih_ref, whh_ref, bih_ref, bhh_ref):
    xb = _cat_q(x2_ref)
    msb = _cat_q(ms_ref)
    h = msb / jnp.maximum(cnt_ref[...], 1.0)
    gi = _dot(xb, wih_ref[...]) + bih_ref[...]
    gh = _dot(h, whh_ref[...]) + bhh_ref[...]
    r = jax.nn.sigmoid(gi[:, :H] + gh[:, :H])
    z = jax.nn.sigmoid(gi[:, H:2 * H] + gh[:, H:2 * H])
    n = jnp.tanh(gi[:, 2 * H:] + r * gh[:, 2 * H:])
    hn = (1.0 - z) * n + z * h
    return jnp.where(h == 0.0, xb, hn)


_GRU_SPECS = [
    pl.BlockSpec((NQ, BN, FQ), lambda i: (0, i, 0)),
    pl.BlockSpec((NQ, BN, FQ), lambda i: (0, i, 0)),
    pl.BlockSpec((BN, 1), lambda i: (i, 0)),
    pl.BlockSpec((H, 3 * H), lambda i: (0, 0)),
    pl.BlockSpec((H, 3 * H), lambda i: (0, 0)),
    pl.BlockSpec((1, 3 * H), lambda i: (0, 0)),
    pl.BlockSpec((1, 3 * H), lambda i: (0, 0)),
]


def _gru_plain_body(x2_ref, ms_ref, cnt_ref, wih, whh, bih, bhh, o_ref):
    xo = _gru_core(x2_ref, ms_ref, cnt_ref, wih, whh, bih, bhh)
    _split_q(xo, o_ref)


def _gru_plain(x2, ms, cnt, wiht, whht, bihr, bhhr):
    return pl.pallas_call(
        _gru_plain_body,
        grid=(N // BN,),
        in_specs=_GRU_SPECS,
        out_specs=pl.BlockSpec((NQ, BN, FQ), lambda i: (0, i, 0)),
        out_shape=jax.ShapeDtypeStruct((NQ, N, FQ), _F32),
    )(x2, ms, cnt, wiht, whht, bihr, bhhr)


def _gru_lin2_body(x2_ref, ms_ref, cnt_ref, wih, whh, bih, bhh, w2_ref,
                   b2_ref, o_ref):
    xo = _gru_core(x2_ref, ms_ref, cnt_ref, wih, whh, bih, bhh)
    y = _dot(jnp.maximum(xo, 0.0), w2_ref[...]) + b2_ref[...]
    _split_q(y, o_ref)


def _gru_lin2(x2, ms, cnt, wiht, whht, bihr, bhhr, w2t, b2r):
    return pl.pallas_call(
        _gru_lin2_body,
        grid=(N // BN,),
        in_specs=_GRU_SPECS + [
            pl.BlockSpec((H, H), lambda i: (0, 0)),
            pl.BlockSpec((1, H), lambda i: (0, 0)),
        ],
        out_specs=pl.BlockSpec((NQ, BN, FQ), lambda i: (0, i, 0)),
        out_shape=jax.ShapeDtypeStruct((NQ, N, FQ), _F32),
    )(x2, ms, cnt, wiht, whht, bihr, bhhr, w2t, b2r)


def _gru_pool_body(x2_ref, ms_ref, cnt_ref, wih, whh, bih, bhh, offs_ref,
                   o_ref, acc_s, acc_c):
    i = pl.program_id(0)

    @pl.when(i == 0)
    def _():
        acc_s[...] = jnp.zeros_like(acc_s)
        acc_c[...] = jnp.zeros_like(acc_c)

    xo = _gru_core(x2_ref, ms_ref, cnt_ref, wih, whh, bih, bhh)
    e2 = jnp.maximum(xo, 0.0)
    iot = lax.broadcasted_iota(jnp.int32, (BN, BATCH), 1)
    mask = (offs_ref[...] == iot).astype(_F32)
    acc_s[...] += lax.dot_general(mask, e2, (((0,), (0,)), ((), ())),
                                  precision=_HI, preferred_element_type=_F32)
    acc_c[...] += lax.dot_general(mask, jnp.ones((BN, 1), _F32),
                                  (((0,), (0,)), ((), ())),
                                  precision=_HI, preferred_element_type=_F32)

    @pl.when(i == pl.num_programs(0) - 1)
    def _():
        o_ref[...] = acc_s[...] / jnp.maximum(acc_c[...], 1.0)


def _gru_pool(x2, ms, cnt, offs, wiht, whht, bihr, bhhr):
    return pl.pallas_call(
        _gru_pool_body,
        grid=(N // BN,),
        in_specs=_GRU_SPECS + [pl.BlockSpec((BN, 1), lambda i: (i, 0))],
        out_specs=pl.BlockSpec((BATCH, H), lambda i: (0, 0)),
        out_shape=jax.ShapeDtypeStruct((BATCH, H), _F32),
        scratch_shapes=[
            pltpu.VMEM((BATCH, H), _F32),
            pltpu.VMEM((BATCH, 1), _F32),
        ],
    )(x2, ms, cnt, wiht, whht, bihr, bhhr, offs)


def kernel(x, offset, edge, W1, b1, W2, b2, W_ih, W_hh, b_ih, b_hh):
    edge = edge.astype(jnp.int32)
    offs = offset.astype(jnp.int32).reshape(N, 1)
    w1t = W1.T
    w2t = W2.T
    wiht = W_ih.T
    whht = W_hh.T
    b1r = b1.reshape(1, H)
    b2r = b2.reshape(1, H)
    bihr = b_ih.reshape(1, 3 * H)
    bhhr = b_hh.reshape(1, 3 * H)

    pad = EPAD - E
    src = jnp.concatenate(
        [edge[:, 0, :], jnp.zeros((2, pad), jnp.int32)], axis=1
    ).reshape(2, NT * NB, BLK)
    dst = jnp.concatenate(
        [edge[:, 1, :], jnp.full((2, pad), N, jnp.int32)], axis=1
    ).reshape(2, NT * NB, BLK)

    cnts = _sc_counts(dst)
    cnt0 = cnts[0, :N].reshape(N, 1)
    cnt1 = cnts[1, :N].reshape(N, 1)

    x2 = _lin1(x, w1t, b1r)
    ms = _sc_msum(x2, src[0], dst[0])
    x2 = _gru_plain(x2, ms, cnt0, wiht, whht, bihr, bhhr)
    ms = _sc_msum(x2, src[1], dst[1])
    x2 = _gru_lin2(x2, ms, cnt1, wiht, whht, bihr, bhhr, w2t, b2r)
    ms = _sc_msum(x2, src[0], dst[0])
    x2 = _gru_plain(x2, ms, cnt0, wiht, whht, bihr, bhhr)
    ms = _sc_msum(x2, src[1], dst[1])
    return _gru_pool(x2, ms, cnt1, offs, wiht, whht, bihr, bhhr)


# speedup vs baseline: 8.5949x; 1.3553x over previous
import functools

import jax
import jax.numpy as jnp
from jax import lax
from jax.experimental import pallas as pl
from jax.experimental.pallas import tpu as pltpu
from jax.experimental.pallas import tpu_sc as plsc

N = 50000
E = 800000
IN = 128
H = 64
FQ = 16
NQ = 4
BATCH = 64
NR = N // 2
NTAB = N * NQ
TLEN = NTAB - 3
NACC = 51200
NTAB2 = 4 * NACC

NT = 16
BLK = 128
NB = 392
EPAD = NT * NB * BLK
MAC = 56
NMAC = NB // MAC
RING = 4
ROWS_T = NACC // NT
DCH = ROWS_T // BLK
ZB = 640
NPAD_C = 51200
CT = NPAD_C // NT
ZBC = 640

_F32 = jnp.float32
_HI = lax.Precision.HIGHEST


def _mesh():
    return plsc.VectorSubcoreMesh(core_axis_name="c", subcore_axis_name="s")


def _sc_msum(xt, src4, dst2):
    @functools.partial(
        pl.kernel,
        out_type=jax.ShapeDtypeStruct((NTAB2, FQ), _F32),
        mesh=_mesh(),
        compiler_params=pltpu.CompilerParams(use_tc_tiling_on_sc=False),
        scratch_types=[
            pltpu.VMEM((MAC, BLK), jnp.int32),
            pltpu.VMEM((MAC, BLK), jnp.int32),
            pltpu.VMEM((RING, BLK, FQ), _F32),
            pltpu.VMEM((ZB, FQ), _F32),
            pltpu.VMEM((BLK,), jnp.int32),
            pltpu.VMEM_SHARED((NACC, FQ), _F32),
            pltpu.SemaphoreType.DMA,
            pltpu.SemaphoreType.DMA,
            pltpu.SemaphoreType.DMA,
            pltpu.SemaphoreType.DMA,
        ],
    )
    def k(xt_hbm, src_hbm, dst_hbm, out_hbm, sidx, didx, rows, zbuf, drx,
          acc, sm0, sm1, sm2, sm3):
        c = lax.axis_index("c")
        s = lax.axis_index("s")
        sems = (sm0, sm1, sm2, sm3)

        z16 = jnp.zeros((16,), _F32)
        lane4 = (jnp.arange(16, dtype=jnp.int32) * 4)

        def zfill(i, carry):
            zbuf[i, :] = z16
            return carry

        lax.fori_loop(0, ZB, zfill, 0)

        def one_pass(q):
            base = s * ROWS_T
            for kk in range(ROWS_T // ZB):
                pltpu.sync_copy(zbuf, acc.at[pl.ds(base + kk * ZB, ZB)])

            plsc.subcore_barrier()

            table = xt_hbm.at[pl.ds(q, TLEN)]

            def mac_step(m, carry):
                row0 = s * NB + m * MAC
                pltpu.sync_copy(src_hbm.at[pl.ds(row0, MAC)], sidx)
                pltpu.sync_copy(dst_hbm.at[pl.ds(row0, MAC)], didx)
                for b in range(RING):
                    pltpu.async_copy(table.at[sidx.at[b]], rows.at[b], sems[b])

                def step(g, carry2):
                    for b in range(RING):
                        j = g * RING + b
                        pltpu.make_async_copy(
                            table.at[sidx.at[j]], rows.at[b], sems[b]).wait()
                        pltpu.sync_copy(rows.at[b], acc.at[didx.at[j]],
                                        add=True)

                        @pl.when(g < MAC // RING - 1)
                        def _():
                            pltpu.async_copy(table.at[sidx.at[j + RING]],
                                             rows.at[b], sems[b])
                    return carry2

                lax.fori_loop(0, MAC // RING, step, 0)
                return carry

            lax.fori_loop(0, NMAC, mac_step, 0)
            plsc.subcore_barrier()

            r0 = s * ROWS_T

            def dstep(kk, carry):
                base_row = r0 + kk * BLK
                for i in range(BLK // 16):
                    drx[pl.ds(16 * i, 16)] = (
                        lane4 + (4 * base_row + 64 * i + q))
                pltpu.sync_copy(acc.at[pl.ds(base_row, BLK)], rows.at[0])
                pltpu.sync_copy(rows.at[0], out_hbm.at[drx])
                return carry

            lax.fori_loop(0, DCH, dstep, 0)
            plsc.subcore_barrier()

        @pl.when(c == 0)
        def _():
            one_pass(0)
            one_pass(1)

        @pl.when(c == 1)
        def _():
            one_pass(2)
            one_pass(3)

    return k(xt, src4, dst2)


def _sc_counts(dst2):
    @functools.partial(
        pl.kernel,
        out_type=jax.ShapeDtypeStruct((2, NPAD_C), _F32),
        mesh=_mesh(),
        compiler_params=pltpu.CompilerParams(use_tc_tiling_on_sc=False),
        scratch_types=[
            pltpu.VMEM((MAC, BLK), jnp.int32),
            pltpu.VMEM((BLK,), _F32),
            pltpu.VMEM((ZBC,), _F32),
            pltpu.VMEM_SHARED((NPAD_C,), _F32),
            pltpu.SemaphoreType.DMA,
            pltpu.SemaphoreType.DMA,
            pltpu.SemaphoreType.DMA,
            pltpu.SemaphoreType.DMA,
        ],
    )
    def k(dst_hbm, out_hbm, didx, ones_v, zbuf, cacc, sm0, sm1, sm2, sm3):
        c = lax.axis_index("c")
        s = lax.axis_index("s")
        sems = (sm0, sm1, sm2, sm3)

        one16 = jnp.ones((16,), _F32)
        z16 = jnp.zeros((16,), _F32)
        for i in range(BLK // 16):
            ones_v[pl.ds(16 * i, 16)] = one16

        def zf(i, carry):
            zbuf[pl.ds(i * 16, 16)] = z16
            return carry

        lax.fori_loop(0, ZBC // 16, zf, 0)
        base = s * CT
        for kk in range(CT // ZBC):
            pltpu.sync_copy(zbuf, cacc.at[pl.ds(base + kk * ZBC, ZBC)])
        plsc.subcore_barrier()

        def run(ci):
            def mac_step(m, carry):
                row0 = s * NB + m * MAC
                pltpu.sync_copy(dst_hbm.at[ci].at[pl.ds(row0, MAC)], didx)

                def step(g, carry2):
                    for b in range(RING):
                        j = g * RING + b

                        @pl.when(g > 0)
                        def _():
                            pltpu.make_async_copy(
                                ones_v, cacc.at[didx.at[j]], sems[b]).wait()

                        pltpu.async_copy(ones_v, cacc.at[didx.at[j]], sems[b],
                                         add=True)
                    return carry2

                lax.fori_loop(0, MAC // RING, step, 0)
                for b in range(RING):
                    pltpu.make_async_copy(
                        ones_v, cacc.at[didx.at[b]], sems[b]).wait()
                return carry

            lax.fori_loop(0, NMAC, mac_step, 0)

        @pl.when(c == 0)
        def _():
            run(0)

        @pl.when(c == 1)
        def _():
            run(1)

        plsc.subcore_barrier()
        dr = pl.ds(s * CT, CT)

        @pl.when(c == 0)
        def _():
            pltpu.sync_copy(cacc.at[dr], out_hbm.at[0].at[dr])

        @pl.when(c == 1)
        def _():
            pltpu.sync_copy(cacc.at[dr], out_hbm.at[1].at[dr])

    return k(dst2)


BR = 1000


def _dot(a, b):
    return jnp.dot(a, b, preferred_element_type=_F32, precision=_HI)


def _lin1_body(x_ref, w_ref, b_ref, o_ref):
    o_ref[...] = _dot(x_ref[...], w_ref[...]) + b_ref[...]


def _lin1(xp, w1p, b1p):
    return pl.pallas_call(
        _lin1_body,
        grid=(NR // BR,),
        in_specs=[
            pl.BlockSpec((BR, 2 * IN), lambda i: (i, 0)),
            pl.BlockSpec((2 * IN, IN), lambda i: (0, 0)),
            pl.BlockSpec((1, IN), lambda i: (0, 0)),
        ],
        out_specs=pl.BlockSpec((BR, IN), lambda i: (i, 0)),
        out_shape=jax.ShapeDtypeStruct((NR, IN), _F32),
    )(xp, w1p, b1p)


def _pair(a, b):
    return jnp.concatenate([a, b], axis=1)


def _gru_core(x2_ref, ms_ref, cnt_ref, wih_ref, whh_ref, bih_ref, bhh_ref):
    xb = x2_ref[...]
    msb = ms_ref[...]
    cb = cnt_ref[...]
    cfull = _pair(jnp.broadcast_to(cb[:, 0:1], (BR, H)),
                  jnp.broadcast_to(cb[:, 1:2], (BR, H)))
    h = msb / jnp.maximum(cfull, 1.0)
    gi = _dot(xb, wih_ref[...]) + bih_ref[...]
    gh = _dot(h, whh_ref[...]) + bhh_ref[...]
    ir = _pair(gi[:, 0:H], gi[:, 3 * H:4 * H])
    iz = _pair(gi[:, H:2 * H], gi[:, 4 * H:5 * H])
    inn = _pair(gi[:, 2 * H:3 * H], gi[:, 5 * H:6 * H])
    hr = _pair(gh[:, 0:H], gh[:, 3 * H:4 * H])
    hz = _pair(gh[:, H:2 * H], gh[:, 4 * H:5 * H])
    hn = _pair(gh[:, 2 * H:3 * H], gh[:, 5 * H:6 * H])
    r = jax.nn.sigmoid(ir + hr)
    z = jax.nn.sigmoid(iz + hz)
    n = jnp.tanh(inn + r * hn)
    hnew = (1.0 - z) * n + z * h
    return jnp.where(h == 0.0, xb, hnew)


_GRU_SPECS = [
    pl.BlockSpec((BR, 2 * H), lambda i: (i, 0)),
    pl.BlockSpec((BR, 2 * H), lambda i: (i, 0)),
    pl.BlockSpec((BR, 2), lambda i: (i, 0)),
    pl.BlockSpec((2 * H, 6 * H), lambda i: (0, 0)),
    pl.BlockSpec((2 * H, 6 * H), lambda i: (0, 0)),
    pl.BlockSpec((1, 6 * H), lambda i: (0, 0)),
    pl.BlockSpec((1, 6 * H), lambda i: (0, 0)),
]


def _gru_plain_body(x2_ref, ms_ref, cnt_ref, wih, whh, bih, bhh, o_ref):
    o_ref[...] = _gru_core(x2_ref, ms_ref, cnt_ref, wih, whh, bih, bhh)


def _gru_plain(x2, ms, cnt, wihp, whhp, bihp, bhhp):
    return pl.pallas_call(
        _gru_plain_body,
        grid=(NR // BR,),
        in_specs=_GRU_SPECS,
        out_specs=pl.BlockSpec((BR, 2 * H), lambda i: (i, 0)),
        out_shape=jax.ShapeDtypeStruct((NR, 2 * H), _F32),
    )(x2, ms, cnt, wihp, whhp, bihp, bhhp)


def _gru_lin2_body(x2_ref, ms_ref, cnt_ref, wih, whh, bih, bhh, w2_ref,
                   b2_ref, o_ref):
    xo = _gru_core(x2_ref, ms_ref, cnt_ref, wih, whh, bih, bhh)
    o_ref[...] = _dot(jnp.maximum(xo, 0.0), w2_ref[...]) + b2_ref[...]


def _gru_lin2(x2, ms, cnt, wihp, whhp, bihp, bhhp, w2p, b2p):
    return pl.pallas_call(
        _gru_lin2_body,
        grid=(NR // BR,),
        in_specs=_GRU_SPECS + [
            pl.BlockSpec((2 * H, 2 * H), lambda i: (0, 0)),
            pl.BlockSpec((1, 2 * H), lambda i: (0, 0)),
        ],
        out_specs=pl.BlockSpec((BR, 2 * H), lambda i: (i, 0)),
        out_shape=jax.ShapeDtypeStruct((NR, 2 * H), _F32),
    )(x2, ms, cnt, wihp, whhp, bihp, bhhp, w2p, b2p)


def _gru_pool_body(x2_ref, ms_ref, cnt_ref, wih, whh, bih, bhh, offs_ref,
                   o_ref, acc_s, acc_c):
    i = pl.program_id(0)

    @pl.when(i == 0)
    def _():
        acc_s[...] = jnp.zeros_like(acc_s)
        acc_c[...] = jnp.zeros_like(acc_c)

    xo = _gru_core(x2_ref, ms_ref, cnt_ref, wih, whh, bih, bhh)
    e2 = jnp.maximum(xo, 0.0)
    iot = lax.broadcasted_iota(jnp.int32, (BR, BATCH), 1)
    m_e = (offs_ref[:, 0:1] == iot).astype(_F32)
    m_o = (offs_ref[:, 1:2] == iot).astype(_F32)
    dn = (((0,), (0,)), ((), ()))
    acc_s[...] += (
        lax.dot_general(m_e, e2[:, :H], dn, precision=_HI,
                        preferred_element_type=_F32)
        + lax.dot_general(m_o, e2[:, H:], dn, precision=_HI,
                          preferred_element_type=_F32))
    ones = jnp.ones((BR, 1), _F32)
    acc_c[...] += (
        lax.dot_general(m_e, ones, dn, precision=_HI,
                        preferred_element_type=_F32)
        + lax.dot_general(m_o, ones, dn, precision=_HI,
                          preferred_element_type=_F32))

    @pl.when(i == pl.num_programs(0) - 1)
    def _():
        o_ref[...] = acc_s[...] / jnp.maximum(acc_c[...], 1.0)


def _gru_pool(x2, ms, cnt, offs, wihp, whhp, bihp, bhhp):
    return pl.pallas_call(
        _gru_pool_body,
        grid=(NR // BR,),
        in_specs=_GRU_SPECS + [pl.BlockSpec((BR, 2), lambda i: (i, 0))],
        out_specs=pl.BlockSpec((BATCH, H), lambda i: (0, 0)),
        out_shape=jax.ShapeDtypeStruct((BATCH, H), _F32),
        scratch_shapes=[
            pltpu.VMEM((BATCH, H), _F32),
            pltpu.VMEM((BATCH, 1), _F32),
        ],
    )(x2, ms, cnt, wihp, whhp, bihp, bhhp, offs)


def _blockdiag(w):
    z = jnp.zeros_like(w)
    return jnp.concatenate(
        [jnp.concatenate([w, z], axis=1), jnp.concatenate([z, w], axis=1)],
        axis=0)


def kernel(x, offset, edge, W1, b1, W2, b2, W_ih, W_hh, b_ih, b_hh):
    edge = edge.astype(jnp.int32)
    offs = offset.astype(jnp.int32).reshape(NR, 2)
    w1p = _blockdiag(W1.T)
    w2p = _blockdiag(W2.T)
    wihp = _blockdiag(W_ih.T)
    whhp = _blockdiag(W_hh.T)
    b1p = jnp.tile(b1, 2).reshape(1, 2 * H)
    b2p = jnp.tile(b2, 2).reshape(1, 2 * H)
    bihp = jnp.tile(b_ih, 2).reshape(1, 6 * H)
    bhhp = jnp.tile(b_hh, 2).reshape(1, 6 * H)

    pad = EPAD - E
    src4 = jnp.concatenate(
        [edge[:, 0, :] * 4, jnp.zeros((2, pad), jnp.int32)], axis=1
    ).reshape(2, NT * NB, BLK)
    dst = jnp.concatenate(
        [edge[:, 1, :], jnp.full((2, pad), N, jnp.int32)], axis=1
    ).reshape(2, NT * NB, BLK)

    cnts = _sc_counts(dst)
    cnt0 = cnts[0, :N].reshape(NR, 2)
    cnt1 = cnts[1, :N].reshape(NR, 2)

    xp = x.reshape(NR, 2 * IN)
    x2 = _lin1(xp, w1p, b1p)
    ms = _sc_msum(x2.reshape(NTAB, FQ), src4[0], dst[0]).reshape(-1, 2 * H)
    x2 = _gru_plain(x2, ms, cnt0, wihp, whhp, bihp, bhhp)
    ms = _sc_msum(x2.reshape(NTAB, FQ), src4[1], dst[1]).reshape(-1, 2 * H)
    x2 = _gru_lin2(x2, ms, cnt1, wihp, whhp, bihp, bhhp, w2p, b2p)
    ms = _sc_msum(x2.reshape(NTAB, FQ), src4[0], dst[0]).reshape(-1, 2 * H)
    x2 = _gru_plain(x2, ms, cnt0, wihp, whhp, bihp, bhhp)
    ms = _sc_msum(x2.reshape(NTAB, FQ), src4[1], dst[1]).reshape(-1, 2 * H)
    return _gru_pool(x2, ms, cnt1, offs, wihp, whhp, bihp, bhhp)


# speedup vs baseline: 11.9335x; 1.3884x over previous
import functools

import jax
import jax.numpy as jnp
from jax import lax
from jax.experimental import pallas as pl
from jax.experimental.pallas import tpu as pltpu
from jax.experimental.pallas import tpu_sc as plsc

N = 50000
E = 800000
IN = 128
H = 64
FQ = 16
NQ = 4
BATCH = 64
NR = N // 2
NTAB = N * NQ
TLEN = NTAB - 3
NACC = 51200
NTAB2 = 4 * NACC

NT = 16
BLK = 128
NB = 392
EPAD = NT * NB * BLK
MAC = 56
NMAC = NB // MAC
RING = 4
NBUF = 8
ROWS_T = NACC // NT
DCH = ROWS_T // BLK
ZB = 640
NPAD_C = 51200
CT = NPAD_C // NT
ZBC = 640

_F32 = jnp.float32
_HI = lax.Precision.DEFAULT


def _mesh():
    return plsc.VectorSubcoreMesh(core_axis_name="c", subcore_axis_name="s")


def _sc_msum(xt, src4, dst2):
    @functools.partial(
        pl.kernel,
        out_type=jax.ShapeDtypeStruct((NTAB2, FQ), _F32),
        mesh=_mesh(),
        compiler_params=pltpu.CompilerParams(use_tc_tiling_on_sc=False),
        scratch_types=[
            pltpu.VMEM((2, MAC, BLK), jnp.int32),
            pltpu.VMEM((2, MAC, BLK), jnp.int32),
            pltpu.VMEM((NBUF, BLK, FQ), _F32),
            pltpu.VMEM((ZB, FQ), _F32),
            pltpu.VMEM((2, BLK), jnp.int32),
            pltpu.VMEM_SHARED((NACC, FQ), _F32),
            pltpu.SemaphoreType.DMA((NBUF,)),
            pltpu.SemaphoreType.DMA((NBUF,)),
            pltpu.SemaphoreType.DMA((2,)),
            pltpu.SemaphoreType.DMA((2,)),
        ],
    )
    def k(xt_hbm, src_hbm, dst_hbm, out_hbm, sidx, didx, rows, zbuf, drx,
          acc, gsem, ssem, isem, dsem):
        c = lax.axis_index("c")
        s = lax.axis_index("s")

        z16 = jnp.zeros((16,), _F32)
        lane4 = (jnp.arange(16, dtype=jnp.int32) * 4)

        def zfill(i, carry):
            zbuf[i, :] = z16
            return carry

        lax.fori_loop(0, ZB, zfill, 0)

        def idx_load(m, slot, fire):
            row0 = s * NB + m * MAC
            a = pltpu.make_async_copy(src_hbm.at[pl.ds(row0, MAC)],
                                      sidx.at[slot], isem.at[slot])
            b = pltpu.make_async_copy(dst_hbm.at[pl.ds(row0, MAC)],
                                      didx.at[slot], isem.at[slot])
            if fire:
                a.start()
                b.start()
            else:
                a.wait()
                b.wait()

        def one_pass(q):
            base = s * ROWS_T
            for kk in range(ROWS_T // ZB):
                pltpu.sync_copy(zbuf, acc.at[pl.ds(base + kk * ZB, ZB)])

            plsc.subcore_barrier()

            table = xt_hbm.at[pl.ds(q, TLEN)]

            def run_macro(m, slot):
                sx = sidx.at[slot]
                dx = didx.at[slot]

                def wait_g(u, j):
                    pltpu.make_async_copy(table.at[sx.at[j]], rows.at[u],
                                          gsem.at[u]).wait()

                def fire_g(u, j):
                    pltpu.async_copy(table.at[sx.at[j]], rows.at[u],
                                     gsem.at[u])

                def fire_s(u, j):
                    pltpu.async_copy(rows.at[u], acc.at[dx.at[j]],
                                     ssem.at[u], add=True)

                def wait_s(u, j):
                    pltpu.make_async_copy(rows.at[u], acc.at[dx.at[j]],
                                          ssem.at[u]).wait()

                for u in range(4):
                    fire_g(u, u)

                def slots(g, carry):
                    for u in range(NBUF):
                        j = g * NBUF + u
                        wait_g(u, j)
                        fire_s(u, j)
                        u4 = (u + 4) % NBUF
                        if u < 4:
                            @pl.when(g > 0)
                            def _():
                                wait_s(u4, j)
                            fire_g(u4, j + 4)
                        else:
                            wait_s(u4, j)

                            @pl.when(g < MAC // NBUF - 1)
                            def _():
                                fire_g(u4, j + 4)
                    return carry

                lax.fori_loop(0, MAC // NBUF, slots, 0)
                for u in range(4, NBUF):
                    wait_s(u, MAC - 8 + u)

            idx_load(0, 0, True)
            idx_load(1, 1, True)

            def mpair(p, carry):
                m0 = 2 * p
                idx_load(m0, 0, False)
                run_macro(m0, 0)
                idx_load(m0 + 2, 0, True)
                idx_load(m0 + 1, 1, False)
                run_macro(m0 + 1, 1)

                @pl.when(p < (NMAC - 1) // 2 - 1)
                def _():
                    idx_load(m0 + 3, 1, True)
                return carry

            lax.fori_loop(0, (NMAC - 1) // 2, mpair, 0)
            idx_load(NMAC - 1, 0, False)
            run_macro(NMAC - 1, 0)

            plsc.subcore_barrier()

            r0 = s * ROWS_T

            def dchunk(kk, u, wait_prev):
                base_row = r0 + kk * BLK
                if wait_prev:
                    pltpu.make_async_copy(rows.at[u], out_hbm.at[drx.at[u]],
                                          dsem.at[u]).wait()
                for i in range(BLK // 16):
                    drx[u, pl.ds(16 * i, 16)] = (
                        lane4 + (4 * base_row + 64 * i + q))
                pltpu.sync_copy(acc.at[pl.ds(base_row, BLK)], rows.at[u])
                pltpu.async_copy(rows.at[u], out_hbm.at[drx.at[u]],
                                 dsem.at[u])

            def dpair(p, carry):
                for u in range(2):
                    kk = p * 2 + u

                    @pl.when(p > 0)
                    def _():
                        pltpu.make_async_copy(
                            rows.at[u], out_hbm.at[drx.at[u]],
                            dsem.at[u]).wait()
                    for i in range(BLK // 16):
                        drx[u, pl.ds(16 * i, 16)] = (
                            lane4 + (4 * (r0 + kk * BLK) + 64 * i + q))
                    pltpu.sync_copy(acc.at[pl.ds(r0 + kk * BLK, BLK)],
                                    rows.at[u])
                    pltpu.async_copy(rows.at[u], out_hbm.at[drx.at[u]],
                                     dsem.at[u])
                return carry

            lax.fori_loop(0, DCH // 2, dpair, 0)
            pltpu.make_async_copy(rows.at[0], out_hbm.at[drx.at[0]],
                                  dsem.at[0]).wait()
            for i in range(BLK // 16):
                drx[0, pl.ds(16 * i, 16)] = (
                    lane4 + (4 * (r0 + (DCH - 1) * BLK) + 64 * i + q))
            pltpu.sync_copy(acc.at[pl.ds(r0 + (DCH - 1) * BLK, BLK)],
                            rows.at[0])
            pltpu.async_copy(rows.at[0], out_hbm.at[drx.at[0]], dsem.at[0])
            pltpu.make_async_copy(rows.at[0], out_hbm.at[drx.at[0]],
                                  dsem.at[0]).wait()
            pltpu.make_async_copy(rows.at[1], out_hbm.at[drx.at[1]],
                                  dsem.at[1]).wait()
            plsc.subcore_barrier()

        @pl.when(c == 0)
        def _():
            one_pass(0)
            one_pass(1)

        @pl.when(c == 1)
        def _():
            one_pass(2)
            one_pass(3)

    return k(xt, src4, dst2)


def _sc_counts(dst2):
    @functools.partial(
        pl.kernel,
        out_type=jax.ShapeDtypeStruct((2, NPAD_C), _F32),
        mesh=_mesh(),
        compiler_params=pltpu.CompilerParams(use_tc_tiling_on_sc=False),
        scratch_types=[
            pltpu.VMEM((MAC, BLK), jnp.int32),
            pltpu.VMEM((BLK,), _F32),
            pltpu.VMEM((ZBC,), _F32),
            pltpu.VMEM_SHARED((NPAD_C,), _F32),
            pltpu.SemaphoreType.DMA,
            pltpu.SemaphoreType.DMA,
            pltpu.SemaphoreType.DMA,
            pltpu.SemaphoreType.DMA,
        ],
    )
    def k(dst_hbm, out_hbm, didx, ones_v, zbuf, cacc, sm0, sm1, sm2, sm3):
        c = lax.axis_index("c")
        s = lax.axis_index("s")
        sems = (sm0, sm1, sm2, sm3)

        one16 = jnp.ones((16,), _F32)
        z16 = jnp.zeros((16,), _F32)
        for i in range(BLK // 16):
            ones_v[pl.ds(16 * i, 16)] = one16

        def zf(i, carry):
            zbuf[pl.ds(i * 16, 16)] = z16
            return carry

        lax.fori_loop(0, ZBC // 16, zf, 0)
        base = s * CT
        for kk in range(CT // ZBC):
            pltpu.sync_copy(zbuf, cacc.at[pl.ds(base + kk * ZBC, ZBC)])
        plsc.subcore_barrier()

        def run(ci):
            def mac_step(m, carry):
                row0 = s * NB + m * MAC
                pltpu.sync_copy(dst_hbm.at[ci].at[pl.ds(row0, MAC)], didx)

                def step(g, carry2):
                    for b in range(RING):
                        j = g * RING + b

                        @pl.when(g > 0)
                        def _():
                            pltpu.make_async_copy(
                                ones_v, cacc.at[didx.at[j]], sems[b]).wait()

                        pltpu.async_copy(ones_v, cacc.at[didx.at[j]], sems[b],
                                         add=True)
                    return carry2

                lax.fori_loop(0, MAC // RING, step, 0)
                for b in range(RING):
                    pltpu.make_async_copy(
                        ones_v, cacc.at[didx.at[b]], sems[b]).wait()
                return carry

            lax.fori_loop(0, NMAC, mac_step, 0)

        @pl.when(c == 0)
        def _():
            run(0)

        @pl.when(c == 1)
        def _():
            run(1)

        plsc.subcore_barrier()
        dr = pl.ds(s * CT, CT)

        @pl.when(c == 0)
        def _():
            pltpu.sync_copy(cacc.at[dr], out_hbm.at[0].at[dr])

        @pl.when(c == 1)
        def _():
            pltpu.sync_copy(cacc.at[dr], out_hbm.at[1].at[dr])

    return k(dst2)


BR = 1000


def _dot(a, b):
    return jnp.dot(a, b, preferred_element_type=_F32, precision=_HI)


def _lin1_body(x_ref, w_ref, b_ref, o_ref):
    o_ref[...] = _dot(x_ref[...], w_ref[...]) + b_ref[...]


def _lin1(xp, w1p, b1p):
    return pl.pallas_call(
        _lin1_body,
        grid=(NR // BR,),
        in_specs=[
            pl.BlockSpec((BR, 2 * IN), lambda i: (i, 0)),
            pl.BlockSpec((2 * IN, IN), lambda i: (0, 0)),
            pl.BlockSpec((1, IN), lambda i: (0, 0)),
        ],
        out_specs=pl.BlockSpec((BR, IN), lambda i: (i, 0)),
        out_shape=jax.ShapeDtypeStruct((NR, IN), _F32),
    )(xp, w1p, b1p)


def _pair(a, b):
    return jnp.concatenate([a, b], axis=1)


def _gru_core(x2_ref, ms_ref, cnt_ref, wih_ref, whh_ref, bih_ref, bhh_ref):
    xb = x2_ref[...]
    msb = ms_ref[...]
    cb = cnt_ref[...]
    cfull = _pair(jnp.broadcast_to(cb[:, 0:1], (BR, H)),
                  jnp.broadcast_to(cb[:, 1:2], (BR, H)))
    h = msb / jnp.maximum(cfull, 1.0)
    gi = _dot(xb, wih_ref[...]) + bih_ref[...]
    gh = _dot(h, whh_ref[...]) + bhh_ref[...]
    ir = _pair(gi[:, 0:H], gi[:, 3 * H:4 * H])
    iz = _pair(gi[:, H:2 * H], gi[:, 4 * H:5 * H])
    inn = _pair(gi[:, 2 * H:3 * H], gi[:, 5 * H:6 * H])
    hr = _pair(gh[:, 0:H], gh[:, 3 * H:4 * H])
    hz = _pair(gh[:, H:2 * H], gh[:, 4 * H:5 * H])
    hn = _pair(gh[:, 2 * H:3 * H], gh[:, 5 * H:6 * H])
    r = jax.nn.sigmoid(ir + hr)
    z = jax.nn.sigmoid(iz + hz)
    n = jnp.tanh(inn + r * hn)
    hnew = (1.0 - z) * n + z * h
    return jnp.where(h == 0.0, xb, hnew)


_GRU_SPECS = [
    pl.BlockSpec((BR, 2 * H), lambda i: (i, 0)),
    pl.BlockSpec((BR, 2 * H), lambda i: (i, 0)),
    pl.BlockSpec((BR, 2), lambda i: (i, 0)),
    pl.BlockSpec((2 * H, 6 * H), lambda i: (0, 0)),
    pl.BlockSpec((2 * H, 6 * H), lambda i: (0, 0)),
    pl.BlockSpec((1, 6 * H), lambda i: (0, 0)),
    pl.BlockSpec((1, 6 * H), lambda i: (0, 0)),
]


def _gru_plain_body(x2_ref, ms_ref, cnt_ref, wih, whh, bih, bhh, o_ref):
    o_ref[...] = _gru_core(x2_ref, ms_ref, cnt_ref, wih, whh, bih, bhh)


def _gru_plain(x2, ms, cnt, wihp, whhp, bihp, bhhp):
    return pl.pallas_call(
        _gru_plain_body,
        grid=(NR // BR,),
        in_specs=_GRU_SPECS,
        out_specs=pl.BlockSpec((BR, 2 * H), lambda i: (i, 0)),
        out_shape=jax.ShapeDtypeStruct((NR, 2 * H), _F32),
    )(x2, ms, cnt, wihp, whhp, bihp, bhhp)


def _gru_lin2_body(x2_ref, ms_ref, cnt_ref, wih, whh, bih, bhh, w2_ref,
                   b2_ref, o_ref):
    xo = _gru_core(x2_ref, ms_ref, cnt_ref, wih, whh, bih, bhh)
    o_ref[...] = _dot(jnp.maximum(xo, 0.0), w2_ref[...]) + b2_ref[...]


def _gru_lin2(x2, ms, cnt, wihp, whhp, bihp, bhhp, w2p, b2p):
    return pl.pallas_call(
        _gru_lin2_body,
        grid=(NR // BR,),
        in_specs=_GRU_SPECS + [
            pl.BlockSpec((2 * H, 2 * H), lambda i: (0, 0)),
            pl.BlockSpec((1, 2 * H), lambda i: (0, 0)),
        ],
        out_specs=pl.BlockSpec((BR, 2 * H), lambda i: (i, 0)),
        out_shape=jax.ShapeDtypeStruct((NR, 2 * H), _F32),
    )(x2, ms, cnt, wihp, whhp, bihp, bhhp, w2p, b2p)


def _gru_pool_body(x2_ref, ms_ref, cnt_ref, wih, whh, bih, bhh, offs_ref,
                   o_ref, acc_s, acc_c):
    i = pl.program_id(0)

    @pl.when(i == 0)
    def _():
        acc_s[...] = jnp.zeros_like(acc_s)
        acc_c[...] = jnp.zeros_like(acc_c)

    xo = _gru_core(x2_ref, ms_ref, cnt_ref, wih, whh, bih, bhh)
    e2 = jnp.maximum(xo, 0.0)
    iot = lax.broadcasted_iota(jnp.int32, (BR, BATCH), 1)
    m_e = (offs_ref[:, 0:1] == iot).astype(_F32)
    m_o = (offs_ref[:, 1:2] == iot).astype(_F32)
    dn = (((0,), (0,)), ((), ()))
    acc_s[...] += (
        lax.dot_general(m_e, e2[:, :H], dn, precision=_HI,
                        preferred_element_type=_F32)
        + lax.dot_general(m_o, e2[:, H:], dn, precision=_HI,
                          preferred_element_type=_F32))
    ones = jnp.ones((BR, 1), _F32)
    acc_c[...] += (
        lax.dot_general(m_e, ones, dn, precision=_HI,
                        preferred_element_type=_F32)
        + lax.dot_general(m_o, ones, dn, precision=_HI,
                          preferred_element_type=_F32))

    @pl.when(i == pl.num_programs(0) - 1)
    def _():
        o_ref[...] = acc_s[...] / jnp.maximum(acc_c[...], 1.0)


def _gru_pool(x2, ms, cnt, offs, wihp, whhp, bihp, bhhp):
    return pl.pallas_call(
        _gru_pool_body,
        grid=(NR // BR,),
        in_specs=_GRU_SPECS + [pl.BlockSpec((BR, 2), lambda i: (i, 0))],
        out_specs=pl.BlockSpec((BATCH, H), lambda i: (0, 0)),
        out_shape=jax.ShapeDtypeStruct((BATCH, H), _F32),
        scratch_shapes=[
            pltpu.VMEM((BATCH, H), _F32),
            pltpu.VMEM((BATCH, 1), _F32),
        ],
    )(x2, ms, cnt, wihp, whhp, bihp, bhhp, offs)


def _blockdiag(w):
    z = jnp.zeros_like(w)
    return jnp.concatenate(
        [jnp.concatenate([w, z], axis=1), jnp.concatenate([z, w], axis=1)],
        axis=0)


def kernel(x, offset, edge, W1, b1, W2, b2, W_ih, W_hh, b_ih, b_hh):
    edge = edge.astype(jnp.int32)
    offs = offset.astype(jnp.int32).reshape(NR, 2)
    w1p = _blockdiag(W1.T)
    w2p = _blockdiag(W2.T)
    wihp = _blockdiag(W_ih.T)
    whhp = _blockdiag(W_hh.T)
    b1p = jnp.tile(b1, 2).reshape(1, 2 * H)
    b2p = jnp.tile(b2, 2).reshape(1, 2 * H)
    bihp = jnp.tile(b_ih, 2).reshape(1, 6 * H)
    bhhp = jnp.tile(b_hh, 2).reshape(1, 6 * H)

    pad = EPAD - E
    src4 = jnp.concatenate(
        [edge[:, 0, :] * 4, jnp.zeros((2, pad), jnp.int32)], axis=1
    ).reshape(2, NT * NB, BLK)
    dst = jnp.concatenate(
        [edge[:, 1, :], jnp.full((2, pad), N, jnp.int32)], axis=1
    ).reshape(2, NT * NB, BLK)

    cnts = _sc_counts(dst)
    cnt0 = cnts[0, :N].reshape(NR, 2)
    cnt1 = cnts[1, :N].reshape(NR, 2)

    xp = x.reshape(NR, 2 * IN)
    x2 = _lin1(xp, w1p, b1p)
    ms = _sc_msum(x2.reshape(NTAB, FQ), src4[0], dst[0]).reshape(-1, 2 * H)
    x2 = _gru_plain(x2, ms, cnt0, wihp, whhp, bihp, bhhp)
    ms = _sc_msum(x2.reshape(NTAB, FQ), src4[1], dst[1]).reshape(-1, 2 * H)
    x2 = _gru_lin2(x2, ms, cnt1, wihp, whhp, bihp, bhhp, w2p, b2p)
    ms = _sc_msum(x2.reshape(NTAB, FQ), src4[0], dst[0]).reshape(-1, 2 * H)
    x2 = _gru_plain(x2, ms, cnt0, wihp, whhp, bihp, bhhp)
    ms = _sc_msum(x2.reshape(NTAB, FQ), src4[1], dst[1]).reshape(-1, 2 * H)
    return _gru_pool(x2, ms, cnt1, offs, wihp, whhp, bihp, bhhp)


# speedup vs baseline: 13.9931x; 1.1726x over previous
import functools

import jax
import jax.numpy as jnp
from jax import lax
from jax.experimental import pallas as pl
from jax.experimental.pallas import tpu as pltpu
from jax.experimental.pallas import tpu_sc as plsc

N = 50000
E = 800000
IN = 128
H = 64
FQ = 16
NQ = 4
BATCH = 64
NR = N // 2
FH = 32
NTABH = 2 * N
TLENH = NTABH - 1
NACC = 51200
NTABH2 = 2 * NACC

NT = 16
BLK = 128
NB = 392
EPAD = NT * NB * BLK
MAC = 56
NMAC = NB // MAC
RING = 4
NBUF = 8
ROWS_T = NACC // NT
DCH = ROWS_T // BLK
ZB = 640
NPAD_C = 51200
CT = NPAD_C // NT
ZBC = 640

_F32 = jnp.float32
_BF16 = jnp.bfloat16
_HI = lax.Precision.DEFAULT


def _mesh():
    return plsc.VectorSubcoreMesh(core_axis_name="c", subcore_axis_name="s")


def _sc_msum(xt, src2, dst2):
    @functools.partial(
        pl.kernel,
        out_type=jax.ShapeDtypeStruct((NTABH2, FH), _BF16),
        mesh=_mesh(),
        compiler_params=pltpu.CompilerParams(use_tc_tiling_on_sc=False),
        scratch_types=[
            pltpu.VMEM((2, MAC, BLK), jnp.int32),
            pltpu.VMEM((2, MAC, BLK), jnp.int32),
            pltpu.VMEM((NBUF, BLK, FH), _BF16),
            pltpu.VMEM((ZB, FH), _BF16),
            pltpu.VMEM((2, BLK), jnp.int32),
            pltpu.VMEM_SHARED((NACC, FH), _BF16),
            pltpu.SemaphoreType.DMA((NBUF,)),
            pltpu.SemaphoreType.DMA((NBUF,)),
            pltpu.SemaphoreType.DMA((2,)),
            pltpu.SemaphoreType.DMA((2,)),
        ],
    )
    def k(xt_hbm, src_hbm, dst_hbm, out_hbm, sidx, didx, rows, zbuf, drx,
          acc, gsem, ssem, isem, dsem):
        c = lax.axis_index("c")
        s = lax.axis_index("s")

        z32 = jnp.zeros((32,), _BF16)
        lane2 = (jnp.arange(16, dtype=jnp.int32) * 2)

        def zfill(i, carry):
            zbuf[i, :] = z32
            return carry

        lax.fori_loop(0, ZB, zfill, 0)

        def idx_load(m, slot, fire):
            row0 = s * NB + m * MAC
            a = pltpu.make_async_copy(src_hbm.at[pl.ds(row0, MAC)],
                                      sidx.at[slot], isem.at[slot])
            b = pltpu.make_async_copy(dst_hbm.at[pl.ds(row0, MAC)],
                                      didx.at[slot], isem.at[slot])
            if fire:
                a.start()
                b.start()
            else:
                a.wait()
                b.wait()

        def one_pass(q):
            base = s * ROWS_T
            for kk in range(ROWS_T // ZB):
                pltpu.sync_copy(zbuf, acc.at[pl.ds(base + kk * ZB, ZB)])

            plsc.subcore_barrier()

            table = xt_hbm.at[pl.ds(q, TLENH)]

            def run_macro(m, slot):
                sx = sidx.at[slot]
                dx = didx.at[slot]

                def wait_g(u, j):
                    pltpu.make_async_copy(table.at[sx.at[j]], rows.at[u],
                                          gsem.at[u]).wait()

                def fire_g(u, j):
                    pltpu.async_copy(table.at[sx.at[j]], rows.at[u],
                                     gsem.at[u])

                def fire_s(u, j):
                    pltpu.async_copy(rows.at[u], acc.at[dx.at[j]],
                                     ssem.at[u], add=True)

                def wait_s(u, j):
                    pltpu.make_async_copy(rows.at[u], acc.at[dx.at[j]],
                                          ssem.at[u]).wait()

                for u in range(4):
                    fire_g(u, u)

                def slots(g, carry):
                    for u in range(NBUF):
                        j = g * NBUF + u
                        wait_g(u, j)
                        fire_s(u, j)
                        u4 = (u + 4) % NBUF
                        if u < 4:
                            @pl.when(g > 0)
                            def _():
                                wait_s(u4, j)
                            fire_g(u4, j + 4)
                        else:
                            wait_s(u4, j)

                            @pl.when(g < MAC // NBUF - 1)
                            def _():
                                fire_g(u4, j + 4)
                    return carry

                lax.fori_loop(0, MAC // NBUF, slots, 0)
                for u in range(4, NBUF):
                    wait_s(u, MAC - 8 + u)

            idx_load(0, 0, True)
            idx_load(1, 1, True)

            def mpair(p, carry):
                m0 = 2 * p
                idx_load(m0, 0, False)
                run_macro(m0, 0)
                idx_load(m0 + 2, 0, True)
                idx_load(m0 + 1, 1, False)
                run_macro(m0 + 1, 1)

                @pl.when(p < (NMAC - 1) // 2 - 1)
                def _():
                    idx_load(m0 + 3, 1, True)
                return carry

            lax.fori_loop(0, (NMAC - 1) // 2, mpair, 0)
            idx_load(NMAC - 1, 0, False)
            run_macro(NMAC - 1, 0)

            plsc.subcore_barrier()

            r0 = s * ROWS_T

            def dpair(p, carry):
                for u in range(2):
                    kk = p * 2 + u

                    @pl.when(p > 0)
                    def _():
                        pltpu.make_async_copy(
                            rows.at[u], out_hbm.at[drx.at[u]],
                            dsem.at[u]).wait()
                    for i in range(BLK // 16):
                        drx[u, pl.ds(16 * i, 16)] = (
                            lane2 + (2 * (r0 + kk * BLK) + 32 * i + q))
                    pltpu.sync_copy(acc.at[pl.ds(r0 + kk * BLK, BLK)],
                                    rows.at[u])
                    pltpu.async_copy(rows.at[u], out_hbm.at[drx.at[u]],
                                     dsem.at[u])
                return carry

            lax.fori_loop(0, DCH // 2, dpair, 0)
            pltpu.make_async_copy(rows.at[0], out_hbm.at[drx.at[0]],
                                  dsem.at[0]).wait()
            for i in range(BLK // 16):
                drx[0, pl.ds(16 * i, 16)] = (
                    lane2 + (2 * (r0 + (DCH - 1) * BLK) + 32 * i + q))
            pltpu.sync_copy(acc.at[pl.ds(r0 + (DCH - 1) * BLK, BLK)],
                            rows.at[0])
            pltpu.async_copy(rows.at[0], out_hbm.at[drx.at[0]], dsem.at[0])
            pltpu.make_async_copy(rows.at[0], out_hbm.at[drx.at[0]],
                                  dsem.at[0]).wait()
            pltpu.make_async_copy(rows.at[1], out_hbm.at[drx.at[1]],
                                  dsem.at[1]).wait()
            plsc.subcore_barrier()

        @pl.when(c == 0)
        def _():
            one_pass(0)

        @pl.when(c == 1)
        def _():
            one_pass(1)

    return k(xt, src2, dst2)


def _sc_counts(dst2):
    @functools.partial(
        pl.kernel,
        out_type=jax.ShapeDtypeStruct((2, NPAD_C), _F32),
        mesh=_mesh(),
        compiler_params=pltpu.CompilerParams(use_tc_tiling_on_sc=False),
        scratch_types=[
            pltpu.VMEM((MAC, BLK), jnp.int32),
            pltpu.VMEM((BLK,), _F32),
            pltpu.VMEM((ZBC,), _F32),
            pltpu.VMEM_SHARED((NPAD_C,), _F32),
            pltpu.SemaphoreType.DMA,
            pltpu.SemaphoreType.DMA,
            pltpu.SemaphoreType.DMA,
            pltpu.SemaphoreType.DMA,
        ],
    )
    def k(dst_hbm, out_hbm, didx, ones_v, zbuf, cacc, sm0, sm1, sm2, sm3):
        c = lax.axis_index("c")
        s = lax.axis_index("s")
        sems = (sm0, sm1, sm2, sm3)

        one16 = jnp.ones((16,), _F32)
        z16 = jnp.zeros((16,), _F32)
        for i in range(BLK // 16):
            ones_v[pl.ds(16 * i, 16)] = one16

        def zf(i, carry):
            zbuf[pl.ds(i * 16, 16)] = z16
            return carry

        lax.fori_loop(0, ZBC // 16, zf, 0)
        base = s * CT
        for kk in range(CT // ZBC):
            pltpu.sync_copy(zbuf, cacc.at[pl.ds(base + kk * ZBC, ZBC)])
        plsc.subcore_barrier()

        def run(ci):
            def mac_step(m, carry):
                row0 = s * NB + m * MAC
                pltpu.sync_copy(dst_hbm.at[ci].at[pl.ds(row0, MAC)], didx)

                def step(g, carry2):
                    for b in range(RING):
                        j = g * RING + b

                        @pl.when(g > 0)
                        def _():
                            pltpu.make_async_copy(
                                ones_v, cacc.at[didx.at[j]], sems[b]).wait()

                        pltpu.async_copy(ones_v, cacc.at[didx.at[j]], sems[b],
                                         add=True)
                    return carry2

                lax.fori_loop(0, MAC // RING, step, 0)
                for b in range(RING):
                    pltpu.make_async_copy(
                        ones_v, cacc.at[didx.at[b]], sems[b]).wait()
                return carry

            lax.fori_loop(0, NMAC, mac_step, 0)

        @pl.when(c == 0)
        def _():
            run(0)

        @pl.when(c == 1)
        def _():
            run(1)

        plsc.subcore_barrier()
        dr = pl.ds(s * CT, CT)

        @pl.when(c == 0)
        def _():
            pltpu.sync_copy(cacc.at[dr], out_hbm.at[0].at[dr])

        @pl.when(c == 1)
        def _():
            pltpu.sync_copy(cacc.at[dr], out_hbm.at[1].at[dr])

    return k(dst2)


BR = 1000


def _dot(a, b):
    return jnp.dot(a, b, preferred_element_type=_F32, precision=_HI)


def _lin1_body(x_ref, w_ref, b_ref, o_ref, ob_ref):
    y = _dot(x_ref[...], w_ref[...]) + b_ref[...]
    o_ref[...] = y
    ob_ref[...] = y.astype(_BF16)


def _lin1(xp, w1p, b1p):
    return pl.pallas_call(
        _lin1_body,
        grid=(NR // BR,),
        in_specs=[
            pl.BlockSpec((BR, 2 * IN), lambda i: (i, 0)),
            pl.BlockSpec((2 * IN, IN), lambda i: (0, 0)),
            pl.BlockSpec((1, IN), lambda i: (0, 0)),
        ],
        out_specs=[
            pl.BlockSpec((BR, IN), lambda i: (i, 0)),
            pl.BlockSpec((BR, IN), lambda i: (i, 0)),
        ],
        out_shape=[
            jax.ShapeDtypeStruct((NR, IN), _F32),
            jax.ShapeDtypeStruct((NR, IN), _BF16),
        ],
    )(xp, w1p, b1p)


def _pair(a, b):
    return jnp.concatenate([a, b], axis=1)


def _gru_core(x2_ref, ms_ref, cnt_ref, wih_ref, whh_ref, bih_ref, bhh_ref):
    xb = x2_ref[...]
    msb = ms_ref[...].astype(_F32)
    cb = cnt_ref[...]
    cfull = _pair(jnp.broadcast_to(cb[:, 0:1], (BR, H)),
                  jnp.broadcast_to(cb[:, 1:2], (BR, H)))
    h = msb / jnp.maximum(cfull, 1.0)
    gi = _dot(xb, wih_ref[...]) + bih_ref[...]
    gh = _dot(h, whh_ref[...]) + bhh_ref[...]
    ir = _pair(gi[:, 0:H], gi[:, 3 * H:4 * H])
    iz = _pair(gi[:, H:2 * H], gi[:, 4 * H:5 * H])
    inn = _pair(gi[:, 2 * H:3 * H], gi[:, 5 * H:6 * H])
    hr = _pair(gh[:, 0:H], gh[:, 3 * H:4 * H])
    hz = _pair(gh[:, H:2 * H], gh[:, 4 * H:5 * H])
    hn = _pair(gh[:, 2 * H:3 * H], gh[:, 5 * H:6 * H])
    r = jax.nn.sigmoid(ir + hr)
    z = jax.nn.sigmoid(iz + hz)
    n = jnp.tanh(inn + r * hn)
    hnew = (1.0 - z) * n + z * h
    return jnp.where(h == 0.0, xb, hnew)


_GRU_SPECS = [
    pl.BlockSpec((BR, 2 * H), lambda i: (i, 0)),
    pl.BlockSpec((BR, 2 * H), lambda i: (i, 0)),
    pl.BlockSpec((BR, 2), lambda i: (i, 0)),
    pl.BlockSpec((2 * H, 6 * H), lambda i: (0, 0)),
    pl.BlockSpec((2 * H, 6 * H), lambda i: (0, 0)),
    pl.BlockSpec((1, 6 * H), lambda i: (0, 0)),
    pl.BlockSpec((1, 6 * H), lambda i: (0, 0)),
]


_DUAL_OUT_SPECS = [
    pl.BlockSpec((BR, 2 * H), lambda i: (i, 0)),
    pl.BlockSpec((BR, 2 * H), lambda i: (i, 0)),
]
_DUAL_OUT_SHAPE = [
    jax.ShapeDtypeStruct((NR, 2 * H), _F32),
    jax.ShapeDtypeStruct((NR, 2 * H), _BF16),
]


def _gru_plain_body(x2_ref, ms_ref, cnt_ref, wih, whh, bih, bhh, o_ref,
                    ob_ref):
    xo = _gru_core(x2_ref, ms_ref, cnt_ref, wih, whh, bih, bhh)
    o_ref[...] = xo
    ob_ref[...] = xo.astype(_BF16)


def _gru_plain(x2, ms, cnt, wihp, whhp, bihp, bhhp):
    return pl.pallas_call(
        _gru_plain_body,
        grid=(NR // BR,),
        in_specs=_GRU_SPECS,
        out_specs=_DUAL_OUT_SPECS,
        out_shape=_DUAL_OUT_SHAPE,
    )(x2, ms, cnt, wihp, whhp, bihp, bhhp)


def _gru_lin2_body(x2_ref, ms_ref, cnt_ref, wih, whh, bih, bhh, w2_ref,
                   b2_ref, o_ref, ob_ref):
    xo = _gru_core(x2_ref, ms_ref, cnt_ref, wih, whh, bih, bhh)
    y = _dot(jnp.maximum(xo, 0.0), w2_ref[...]) + b2_ref[...]
    o_ref[...] = y
    ob_ref[...] = y.astype(_BF16)


def _gru_lin2(x2, ms, cnt, wihp, whhp, bihp, bhhp, w2p, b2p):
    return pl.pallas_call(
        _gru_lin2_body,
        grid=(NR // BR,),
        in_specs=_GRU_SPECS + [
            pl.BlockSpec((2 * H, 2 * H), lambda i: (0, 0)),
            pl.BlockSpec((1, 2 * H), lambda i: (0, 0)),
        ],
        out_specs=_DUAL_OUT_SPECS,
        out_shape=_DUAL_OUT_SHAPE,
    )(x2, ms, cnt, wihp, whhp, bihp, bhhp, w2p, b2p)


def _gru_pool_body(x2_ref, ms_ref, cnt_ref, wih, whh, bih, bhh, offs_ref,
                   o_ref, acc_s, acc_c):
    i = pl.program_id(0)

    @pl.when(i == 0)
    def _():
        acc_s[...] = jnp.zeros_like(acc_s)
        acc_c[...] = jnp.zeros_like(acc_c)

    xo = _gru_core(x2_ref, ms_ref, cnt_ref, wih, whh, bih, bhh)
    e2 = jnp.maximum(xo, 0.0)
    iot = lax.broadcasted_iota(jnp.int32, (BR, BATCH), 1)
    m_e = (offs_ref[:, 0:1] == iot).astype(_F32)
    m_o = (offs_ref[:, 1:2] == iot).astype(_F32)
    dn = (((0,), (0,)), ((), ()))
    acc_s[...] += (
        lax.dot_general(m_e, e2[:, :H], dn, precision=_HI,
                        preferred_element_type=_F32)
        + lax.dot_general(m_o, e2[:, H:], dn, precision=_HI,
                          preferred_element_type=_F32))
    ones = jnp.ones((BR, 1), _F32)
    acc_c[...] += (
        lax.dot_general(m_e, ones, dn, precision=_HI,
                        preferred_element_type=_F32)
        + lax.dot_general(m_o, ones, dn, precision=_HI,
                          preferred_element_type=_F32))

    @pl.when(i == pl.num_programs(0) - 1)
    def _():
        o_ref[...] = acc_s[...] / jnp.maximum(acc_c[...], 1.0)


def _gru_pool(x2, ms, cnt, offs, wihp, whhp, bihp, bhhp):
    return pl.pallas_call(
        _gru_pool_body,
        grid=(NR // BR,),
        in_specs=_GRU_SPECS + [pl.BlockSpec((BR, 2), lambda i: (i, 0))],
        out_specs=pl.BlockSpec((BATCH, H), lambda i: (0, 0)),
        out_shape=jax.ShapeDtypeStruct((BATCH, H), _F32),
        scratch_shapes=[
            pltpu.VMEM((BATCH, H), _F32),
            pltpu.VMEM((BATCH, 1), _F32),
        ],
    )(x2, ms, cnt, wihp, whhp, bihp, bhhp, offs)


def _blockdiag(w):
    z = jnp.zeros_like(w)
    return jnp.concatenate(
        [jnp.concatenate([w, z], axis=1), jnp.concatenate([z, w], axis=1)],
        axis=0)


def kernel(x, offset, edge, W1, b1, W2, b2, W_ih, W_hh, b_ih, b_hh):
    edge = edge.astype(jnp.int32)
    offs = offset.astype(jnp.int32).reshape(NR, 2)
    w1p = _blockdiag(W1.T)
    w2p = _blockdiag(W2.T)
    wihp = _blockdiag(W_ih.T)
    whhp = _blockdiag(W_hh.T)
    b1p = jnp.tile(b1, 2).reshape(1, 2 * H)
    b2p = jnp.tile(b2, 2).reshape(1, 2 * H)
    bihp = jnp.tile(b_ih, 2).reshape(1, 6 * H)
    bhhp = jnp.tile(b_hh, 2).reshape(1, 6 * H)

    pad = EPAD - E
    src2 = jnp.concatenate(
        [edge[:, 0, :] * 2, jnp.zeros((2, pad), jnp.int32)], axis=1
    ).reshape(2, NT * NB, BLK)
    dst = jnp.concatenate(
        [edge[:, 1, :], jnp.full((2, pad), N, jnp.int32)], axis=1
    ).reshape(2, NT * NB, BLK)

    cnts = _sc_counts(dst)
    cnt0 = cnts[0, :N].reshape(NR, 2)
    cnt1 = cnts[1, :N].reshape(NR, 2)

    xp = x.reshape(NR, 2 * IN)
    x2, x2b = _lin1(xp, w1p, b1p)
    ms = _sc_msum(x2b.reshape(NTABH, FH), src2[0], dst[0]).reshape(-1, 2 * H)
    x2, x2b = _gru_plain(x2, ms, cnt0, wihp, whhp, bihp, bhhp)
    ms = _sc_msum(x2b.reshape(NTABH, FH), src2[1], dst[1]).reshape(-1, 2 * H)
    x2, x2b = _gru_lin2(x2, ms, cnt1, wihp, whhp, bihp, bhhp, w2p, b2p)
    ms = _sc_msum(x2b.reshape(NTABH, FH), src2[0], dst[0]).reshape(-1, 2 * H)
    x2, x2b = _gru_plain(x2, ms, cnt0, wihp, whhp, bihp, bhhp)
    ms = _sc_msum(x2b.reshape(NTABH, FH), src2[1], dst[1]).reshape(-1, 2 * H)
    return _gru_pool(x2, ms, cnt1, offs, wihp, whhp, bihp, bhhp)


# speedup vs baseline: 14.1485x; 1.0111x over previous
import functools

import jax
import jax.numpy as jnp
from jax import lax
from jax.experimental import pallas as pl
from jax.experimental.pallas import tpu as pltpu
from jax.experimental.pallas import tpu_sc as plsc

N = 50000
E = 800000
IN = 128
H = 64
FQ = 16
NQ = 4
BATCH = 64
NR = N // 2
NRP = 25024
FH = 32
NTABH = 2 * N
TLENH = NTABH - 1
NACC = 51200
NTABH2 = 2 * NACC

NT = 16
BLK = 128
NB = 392
EPAD = NT * NB * BLK
MAC = 56
NMAC = NB // MAC
RING = 4
NBUF = 8
ROWS_T = NACC // NT
DCH = ROWS_T // BLK
ZB = 640
NPAD_C = 51200
CT = NPAD_C // NT
ZBC = 640

_F32 = jnp.float32
_BF16 = jnp.bfloat16
_HI = lax.Precision.DEFAULT


def _mesh():
    return plsc.VectorSubcoreMesh(core_axis_name="c", subcore_axis_name="s")


def _sc_msum(xt, src2, dst2):
    @functools.partial(
        pl.kernel,
        out_type=jax.ShapeDtypeStruct((NTABH2, FH), _BF16),
        mesh=_mesh(),
        compiler_params=pltpu.CompilerParams(use_tc_tiling_on_sc=False),
        scratch_types=[
            pltpu.VMEM((2, MAC, BLK), jnp.int32),
            pltpu.VMEM((2, MAC, BLK), jnp.int32),
            pltpu.VMEM((NBUF, BLK, FH), _BF16),
            pltpu.VMEM((ZB, FH), _BF16),
            pltpu.VMEM((2, BLK), jnp.int32),
            pltpu.VMEM_SHARED((NACC, FH), _BF16),
            pltpu.SemaphoreType.DMA((NBUF,)),
            pltpu.SemaphoreType.DMA((NBUF,)),
            pltpu.SemaphoreType.DMA((2,)),
            pltpu.SemaphoreType.DMA((2,)),
        ],
    )
    def k(xt_hbm, src_hbm, dst_hbm, out_hbm, sidx, didx, rows, zbuf, drx,
          acc, gsem, ssem, isem, dsem):
        c = lax.axis_index("c")
        s = lax.axis_index("s")

        z32 = jnp.zeros((32,), _BF16)
        lane2 = (jnp.arange(16, dtype=jnp.int32) * 2)

        def zfill(i, carry):
            zbuf[i, :] = z32
            return carry

        lax.fori_loop(0, ZB, zfill, 0)

        def idx_load(m, slot, fire):
            row0 = s * NB + m * MAC
            a = pltpu.make_async_copy(src_hbm.at[pl.ds(row0, MAC)],
                                      sidx.at[slot], isem.at[slot])
            b = pltpu.make_async_copy(dst_hbm.at[pl.ds(row0, MAC)],
                                      didx.at[slot], isem.at[slot])
            if fire:
                a.start()
                b.start()
            else:
                a.wait()
                b.wait()

        def one_pass(q):
            base = s * ROWS_T
            for kk in range(ROWS_T // ZB):
                pltpu.sync_copy(zbuf, acc.at[pl.ds(base + kk * ZB, ZB)])

            plsc.subcore_barrier()

            table = xt_hbm.at[pl.ds(q, TLENH)]

            def run_macro(m, slot):
                sx = sidx.at[slot]
                dx = didx.at[slot]

                def wait_g(u, j):
                    pltpu.make_async_copy(table.at[sx.at[j]], rows.at[u],
                                          gsem.at[u]).wait()

                def fire_g(u, j):
                    pltpu.async_copy(table.at[sx.at[j]], rows.at[u],
                                     gsem.at[u])

                def fire_s(u, j):
                    pltpu.async_copy(rows.at[u], acc.at[dx.at[j]],
                                     ssem.at[u], add=True)

                def wait_s(u, j):
                    pltpu.make_async_copy(rows.at[u], acc.at[dx.at[j]],
                                          ssem.at[u]).wait()

                for u in range(4):
                    fire_g(u, u)

                def slots(g, carry):
                    for u in range(NBUF):
                        j = g * NBUF + u
                        wait_g(u, j)
                        fire_s(u, j)
                        u4 = (u + 4) % NBUF
                        if u < 4:
                            @pl.when(g > 0)
                            def _():
                                wait_s(u4, j)
                            fire_g(u4, j + 4)
                        else:
                            wait_s(u4, j)

                            @pl.when(g < MAC // NBUF - 1)
                            def _():
                                fire_g(u4, j + 4)
                    return carry

                lax.fori_loop(0, MAC // NBUF, slots, 0)
                for u in range(4, NBUF):
                    wait_s(u, MAC - 8 + u)

            idx_load(0, 0, True)
            idx_load(1, 1, True)

            def mpair(p, carry):
                m0 = 2 * p
                idx_load(m0, 0, False)
                run_macro(m0, 0)
                idx_load(m0 + 2, 0, True)
                idx_load(m0 + 1, 1, False)
                run_macro(m0 + 1, 1)

                @pl.when(p < (NMAC - 1) // 2 - 1)
                def _():
                    idx_load(m0 + 3, 1, True)
                return carry

            lax.fori_loop(0, (NMAC - 1) // 2, mpair, 0)
            idx_load(NMAC - 1, 0, False)
            run_macro(NMAC - 1, 0)

            plsc.subcore_barrier()

            r0 = s * ROWS_T

            def dpair(p, carry):
                for u in range(2):
                    kk = p * 2 + u

                    @pl.when(p > 0)
                    def _():
                        pltpu.make_async_copy(
                            rows.at[u], out_hbm.at[drx.at[u]],
                            dsem.at[u]).wait()
                    for i in range(BLK // 16):
                        drx[u, pl.ds(16 * i, 16)] = (
                            lane2 + (2 * (r0 + kk * BLK) + 32 * i + q))
                    pltpu.sync_copy(acc.at[pl.ds(r0 + kk * BLK, BLK)],
                                    rows.at[u])
                    pltpu.async_copy(rows.at[u], out_hbm.at[drx.at[u]],
                                     dsem.at[u])
                return carry

            lax.fori_loop(0, DCH // 2, dpair, 0)
            pltpu.make_async_copy(rows.at[0], out_hbm.at[drx.at[0]],
                                  dsem.at[0]).wait()
            for i in range(BLK // 16):
                drx[0, pl.ds(16 * i, 16)] = (
                    lane2 + (2 * (r0 + (DCH - 1) * BLK) + 32 * i + q))
            pltpu.sync_copy(acc.at[pl.ds(r0 + (DCH - 1) * BLK, BLK)],
                            rows.at[0])
            pltpu.async_copy(rows.at[0], out_hbm.at[drx.at[0]], dsem.at[0])
            pltpu.make_async_copy(rows.at[0], out_hbm.at[drx.at[0]],
                                  dsem.at[0]).wait()
            pltpu.make_async_copy(rows.at[1], out_hbm.at[drx.at[1]],
                                  dsem.at[1]).wait()
            plsc.subcore_barrier()

        @pl.when(c == 0)
        def _():
            one_pass(0)

        @pl.when(c == 1)
        def _():
            one_pass(1)

    return k(xt, src2, dst2)


def _sc_counts(dst2):
    @functools.partial(
        pl.kernel,
        out_type=jax.ShapeDtypeStruct((2, NPAD_C), _F32),
        mesh=_mesh(),
        compiler_params=pltpu.CompilerParams(use_tc_tiling_on_sc=False),
        scratch_types=[
            pltpu.VMEM((MAC, BLK), jnp.int32),
            pltpu.VMEM((BLK,), _F32),
            pltpu.VMEM((ZBC,), _F32),
            pltpu.VMEM_SHARED((NPAD_C,), _F32),
            pltpu.SemaphoreType.DMA,
            pltpu.SemaphoreType.DMA,
            pltpu.SemaphoreType.DMA,
            pltpu.SemaphoreType.DMA,
        ],
    )
    def k(dst_hbm, out_hbm, didx, ones_v, zbuf, cacc, sm0, sm1, sm2, sm3):
        c = lax.axis_index("c")
        s = lax.axis_index("s")
        sems = (sm0, sm1, sm2, sm3)

        one16 = jnp.ones((16,), _F32)
        z16 = jnp.zeros((16,), _F32)
        for i in range(BLK // 16):
            ones_v[pl.ds(16 * i, 16)] = one16

        def zf(i, carry):
            zbuf[pl.ds(i * 16, 16)] = z16
            return carry

        lax.fori_loop(0, ZBC // 16, zf, 0)
        base = s * CT
        for kk in range(CT // ZBC):
            pltpu.sync_copy(zbuf, cacc.at[pl.ds(base + kk * ZBC, ZBC)])
        plsc.subcore_barrier()

        def run(ci):
            def mac_step(m, carry):
                row0 = s * NB + m * MAC
                pltpu.sync_copy(dst_hbm.at[ci].at[pl.ds(row0, MAC)], didx)

                def step(g, carry2):
                    for b in range(RING):
                        j = g * RING + b

                        @pl.when(g > 0)
                        def _():
                            pltpu.make_async_copy(
                                ones_v, cacc.at[didx.at[j]], sems[b]).wait()

                        pltpu.async_copy(ones_v, cacc.at[didx.at[j]], sems[b],
                                         add=True)
                    return carry2

                lax.fori_loop(0, MAC // RING, step, 0)
                for b in range(RING):
                    pltpu.make_async_copy(
                        ones_v, cacc.at[didx.at[b]], sems[b]).wait()
                return carry

            lax.fori_loop(0, NMAC, mac_step, 0)

        @pl.when(c == 0)
        def _():
            run(0)

        @pl.when(c == 1)
        def _():
            run(1)

        plsc.subcore_barrier()
        dr = pl.ds(s * CT, CT)

        @pl.when(c == 0)
        def _():
            pltpu.sync_copy(cacc.at[dr], out_hbm.at[0].at[dr])

        @pl.when(c == 1)
        def _():
            pltpu.sync_copy(cacc.at[dr], out_hbm.at[1].at[dr])

    return k(dst2)


BR = 1000


def _dot(a, b):
    return jnp.dot(a, b, preferred_element_type=_F32, precision=_HI)


def _lin1_body(x_ref, w_ref, b_ref, o_ref, ob_ref):
    y = _dot(x_ref[...], w_ref[...]) + b_ref[...]
    o_ref[...] = y
    ob_ref[...] = y.astype(_BF16)


def _lin1(xp, w1p, b1p):
    return pl.pallas_call(
        _lin1_body,
        grid=(NR // BR,),
        in_specs=[
            pl.BlockSpec((BR, 2 * IN), lambda i: (i, 0)),
            pl.BlockSpec((2 * IN, IN), lambda i: (0, 0)),
            pl.BlockSpec((1, IN), lambda i: (0, 0)),
        ],
        out_specs=[
            pl.BlockSpec((BR, IN), lambda i: (i, 0)),
            pl.BlockSpec((BR, IN), lambda i: (i, 0)),
        ],
        out_shape=[
            jax.ShapeDtypeStruct((NR, IN), _F32),
            jax.ShapeDtypeStruct((NRP, IN), _BF16),
        ],
    )(xp, w1p, b1p)


def _pair(a, b):
    return jnp.concatenate([a, b], axis=1)


def _gru_core(x2_ref, ms_ref, cnt_ref, wih_ref, whh_ref, bih_ref, bhh_ref):
    xb = x2_ref[...]
    msb = ms_ref[...].astype(_F32)
    cb = cnt_ref[...]
    cfull = _pair(jnp.broadcast_to(cb[:, 0:1], (BR, H)),
                  jnp.broadcast_to(cb[:, 1:2], (BR, H)))
    h = msb / jnp.maximum(cfull, 1.0)
    gi = _dot(xb, wih_ref[...]) + bih_ref[...]
    gh = _dot(h, whh_ref[...]) + bhh_ref[...]
    ir = _pair(gi[:, 0:H], gi[:, 3 * H:4 * H])
    iz = _pair(gi[:, H:2 * H], gi[:, 4 * H:5 * H])
    inn = _pair(gi[:, 2 * H:3 * H], gi[:, 5 * H:6 * H])
    hr = _pair(gh[:, 0:H], gh[:, 3 * H:4 * H])
    hz = _pair(gh[:, H:2 * H], gh[:, 4 * H:5 * H])
    hn = _pair(gh[:, 2 * H:3 * H], gh[:, 5 * H:6 * H])
    r = jax.nn.sigmoid(ir + hr)
    z = jax.nn.sigmoid(iz + hz)
    n = jnp.tanh(inn + r * hn)
    hnew = (1.0 - z) * n + z * h
    return jnp.where(h == 0.0, xb, hnew)


_GRU_SPECS = [
    pl.BlockSpec((BR, 2 * H), lambda i: (i, 0)),
    pl.BlockSpec((BR, 2 * H), lambda i: (i, 0)),
    pl.BlockSpec((BR, 2), lambda i: (i, 0)),
    pl.BlockSpec((2 * H, 6 * H), lambda i: (0, 0)),
    pl.BlockSpec((2 * H, 6 * H), lambda i: (0, 0)),
    pl.BlockSpec((1, 6 * H), lambda i: (0, 0)),
    pl.BlockSpec((1, 6 * H), lambda i: (0, 0)),
]


_DUAL_OUT_SPECS = [
    pl.BlockSpec((BR, 2 * H), lambda i: (i, 0)),
    pl.BlockSpec((BR, 2 * H), lambda i: (i, 0)),
]
_DUAL_OUT_SHAPE = [
    jax.ShapeDtypeStruct((NR, 2 * H), _F32),
    jax.ShapeDtypeStruct((NRP, 2 * H), _BF16),
]


def _gru_plain_body(x2_ref, ms_ref, cnt_ref, wih, whh, bih, bhh, o_ref,
                    ob_ref):
    xo = _gru_core(x2_ref, ms_ref, cnt_ref, wih, whh, bih, bhh)
    o_ref[...] = xo
    ob_ref[...] = xo.astype(_BF16)


def _gru_plain(x2, ms, cnt, wihp, whhp, bihp, bhhp):
    return pl.pallas_call(
        _gru_plain_body,
        grid=(NR // BR,),
        in_specs=_GRU_SPECS,
        out_specs=_DUAL_OUT_SPECS,
        out_shape=_DUAL_OUT_SHAPE,
    )(x2, ms, cnt, wihp, whhp, bihp, bhhp)


def _gru_lin2_body(x2_ref, ms_ref, cnt_ref, wih, whh, bih, bhh, w2_ref,
                   b2_ref, o_ref, ob_ref):
    xo = _gru_core(x2_ref, ms_ref, cnt_ref, wih, whh, bih, bhh)
    y = _dot(jnp.maximum(xo, 0.0), w2_ref[...]) + b2_ref[...]
    o_ref[...] = y
    ob_ref[...] = y.astype(_BF16)


def _gru_lin2(x2, ms, cnt, wihp, whhp, bihp, bhhp, w2p, b2p):
    return pl.pallas_call(
        _gru_lin2_body,
        grid=(NR // BR,),
        in_specs=_GRU_SPECS + [
            pl.BlockSpec((2 * H, 2 * H), lambda i: (0, 0)),
            pl.BlockSpec((1, 2 * H), lambda i: (0, 0)),
        ],
        out_specs=_DUAL_OUT_SPECS,
        out_shape=_DUAL_OUT_SHAPE,
    )(x2, ms, cnt, wihp, whhp, bihp, bhhp, w2p, b2p)


def _gru_pool_body(x2_ref, ms_ref, cnt_ref, wih, whh, bih, bhh, offs_ref,
                   o_ref, acc_s, acc_c):
    i = pl.program_id(0)

    @pl.when(i == 0)
    def _():
        acc_s[...] = jnp.zeros_like(acc_s)
        acc_c[...] = jnp.zeros_like(acc_c)

    xo = _gru_core(x2_ref, ms_ref, cnt_ref, wih, whh, bih, bhh)
    e2 = jnp.maximum(xo, 0.0)
    iot = lax.broadcasted_iota(jnp.int32, (BR, BATCH), 1)
    m_e = (offs_ref[:, 0:1] == iot).astype(_F32)
    m_o = (offs_ref[:, 1:2] == iot).astype(_F32)
    dn = (((0,), (0,)), ((), ()))
    acc_s[...] += (
        lax.dot_general(m_e, e2[:, :H], dn, precision=_HI,
                        preferred_element_type=_F32)
        + lax.dot_general(m_o, e2[:, H:], dn, precision=_HI,
                          preferred_element_type=_F32))
    ones = jnp.ones((BR, 1), _F32)
    acc_c[...] += (
        lax.dot_general(m_e, ones, dn, precision=_HI,
                        preferred_element_type=_F32)
        + lax.dot_general(m_o, ones, dn, precision=_HI,
                          preferred_element_type=_F32))

    @pl.when(i == pl.num_programs(0) - 1)
    def _():
        o_ref[...] = acc_s[...] / jnp.maximum(acc_c[...], 1.0)


def _gru_pool(x2, ms, cnt, offs, wihp, whhp, bihp, bhhp):
    return pl.pallas_call(
        _gru_pool_body,
        grid=(NR // BR,),
        in_specs=_GRU_SPECS + [pl.BlockSpec((BR, 2), lambda i: (i, 0))],
        out_specs=pl.BlockSpec((BATCH, H), lambda i: (0, 0)),
        out_shape=jax.ShapeDtypeStruct((BATCH, H), _F32),
        scratch_shapes=[
            pltpu.VMEM((BATCH, H), _F32),
            pltpu.VMEM((BATCH, 1), _F32),
        ],
    )(x2, ms, cnt, wihp, whhp, bihp, bhhp, offs)


def _blockdiag(w):
    z = jnp.zeros_like(w)
    return jnp.concatenate(
        [jnp.concatenate([w, z], axis=1), jnp.concatenate([z, w], axis=1)],
        axis=0)


def kernel(x, offset, edge, W1, b1, W2, b2, W_ih, W_hh, b_ih, b_hh):
    edge = edge.astype(jnp.int32)
    offs = offset.astype(jnp.int32).reshape(NR, 2)
    w1p = _blockdiag(W1.T)
    w2p = _blockdiag(W2.T)
    wihp = _blockdiag(W_ih.T)
    whhp = _blockdiag(W_hh.T)
    b1p = jnp.tile(b1, 2).reshape(1, 2 * H)
    b2p = jnp.tile(b2, 2).reshape(1, 2 * H)
    bihp = jnp.tile(b_ih, 2).reshape(1, 6 * H)
    bhhp = jnp.tile(b_hh, 2).reshape(1, 6 * H)

    pad = EPAD - E
    src2 = jnp.concatenate(
        [edge[:, 0, :] * 2, jnp.zeros((2, pad), jnp.int32)], axis=1
    ).reshape(2, NT * NB, BLK)
    dst = jnp.concatenate(
        [edge[:, 1, :], jnp.full((2, pad), N, jnp.int32)], axis=1
    ).reshape(2, NT * NB, BLK)

    cnts = _sc_counts(dst)
    cnt0 = cnts[0, :N].reshape(NR, 2)
    cnt1 = cnts[1, :N].reshape(NR, 2)

    xp = x.reshape(NR, 2 * IN)
    x2, x2b = _lin1(xp, w1p, b1p)
    ms = _sc_msum(x2b.reshape(4 * NRP, FH), src2[0], dst[0]).reshape(-1, 2 * H)
    x2, x2b = _gru_plain(x2, ms, cnt0, wihp, whhp, bihp, bhhp)
    ms = _sc_msum(x2b.reshape(4 * NRP, FH), src2[1], dst[1]).reshape(-1, 2 * H)
    x2, x2b = _gru_lin2(x2, ms, cnt1, wihp, whhp, bihp, bhhp, w2p, b2p)
    ms = _sc_msum(x2b.reshape(4 * NRP, FH), src2[0], dst[0]).reshape(-1, 2 * H)
    x2, x2b = _gru_plain(x2, ms, cnt0, wihp, whhp, bihp, bhhp)
    ms = _sc_msum(x2b.reshape(4 * NRP, FH), src2[1], dst[1]).reshape(-1, 2 * H)
    return _gru_pool(x2, ms, cnt1, offs, wihp, whhp, bihp, bhhp)
